# baseline (speedup 1.0000x reference)
import functools

import jax
import jax.numpy as jnp
from jax.experimental import pallas as pl
from jax.experimental.pallas import tpu as pltpu


def _router_body(H,
                 ui_ref, W1_ref, b1_ref, W2_ref, b2_ref, Wr_ref, br_ref,
                 Wt_ref, bt_ref,
                 Wg1_ref, bg1_ref, Wo1_ref, bo1_ref, Wa1_ref, ba1_ref,
                 Wg2_ref, Wo2_ref, Wa2_ref,
                 w_ref, bias_ref, wf_ref, wg2_ref, wo2_ref, wa2_ref):
    f32 = jnp.float32
    bf16 = jnp.bfloat16
    h = ui_ref[...]
    W1 = W1_ref[...]
    W2 = W2_ref[...]
    b1 = b1_ref[...]
    b2 = b2_ref[...]
    for _ in range(3):
        m = jnp.tanh(jnp.dot(h, W1, preferred_element_type=f32) + b1)
        h = h + jnp.tanh(jnp.dot(m, W2, preferred_element_type=f32) + b2)
    logits = jnp.dot(h, Wr_ref[...], preferred_element_type=f32) + br_ref[...]
    logits = logits - jnp.max(logits, axis=-1, keepdims=True)
    e = jnp.exp(logits)
    w_ref[...] = e / jnp.sum(e, axis=-1, keepdims=True)
    th = jnp.tanh(jnp.dot(h, Wt_ref[...], preferred_element_type=f32) + bt_ref[...])
    pg = jnp.dot(th, Wg1_ref[3:], preferred_element_type=f32) + bg1_ref[...]
    po = jnp.dot(th, Wo1_ref[2:], preferred_element_type=f32) + bo1_ref[...]
    pa = jnp.dot(th, Wa1_ref[1:], preferred_element_type=f32) + ba1_ref[...]
    bias_ref[...] = jnp.concatenate([pg, po, pa], axis=1)
    z = lambda r: jnp.zeros((r, H), f32)
    wg = jnp.concatenate([Wg1_ref[:3], z(5)], axis=0)
    wo = jnp.concatenate([z(3), Wo1_ref[:2], z(3)], axis=0)
    wa = jnp.concatenate([z(5), Wa1_ref[:1], z(2)], axis=0)
    wf_ref[...] = jnp.concatenate([wg, wo, wa], axis=1).astype(bf16)
    wg2_ref[...] = Wg2_ref[...].astype(bf16)
    wo2_ref[...] = Wo2_ref[...].astype(bf16)
    wa2_ref[...] = Wa2_ref[...].astype(bf16)


def _expert_body(TN, H, S,
                 w_ref, bias_ref, wf_ref, wg2_ref, wo2_ref, wa2_ref,
                 bg2_ref, bo2_ref, ba2_ref,
                 Wgs_ref, bgs_ref, Wo3_ref, bo3_ref, Wa3_ref, ba3_ref,
                 c3_ref, c2_ref, c1_ref,
                 sdf_ref, img_ref, aud_ref):
    f32 = jnp.float32
    bf16 = jnp.bfloat16
    b = pl.program_id(0) // S
    w = w_ref[pl.ds(b, 1), :]
    bias = bias_ref[pl.ds(b, 1), :]
    pad = jnp.zeros((TN, 2), bf16)
    cat = jnp.concatenate(
        [c3_ref[0].astype(bf16), c2_ref[0].astype(bf16),
         c1_ref[0].astype(bf16), pad], axis=-1)
    h1 = jnp.dot(cat, wf_ref[...], preferred_element_type=f32)
    h1 = jnp.maximum(h1 + bias, 0.0).astype(bf16)

    hg = jnp.dot(h1[:, :H], wg2_ref[...],
                 preferred_element_type=f32) + bg2_ref[...]
    ho = jnp.dot(h1[:, H:2 * H], wo2_ref[...],
                 preferred_element_type=f32) + bo2_ref[...]
    ha = jnp.dot(h1[:, 2 * H:], wa2_ref[...],
                 preferred_element_type=f32) + ba2_ref[...]
    hg = jnp.maximum(hg, 0.0).astype(bf16)
    ho = jnp.maximum(ho, 0.0).astype(bf16)
    ha = jnp.maximum(ha, 0.0).astype(bf16)

    sdf = jnp.dot(hg, Wgs_ref[...].astype(bf16), preferred_element_type=f32)
    img = jnp.dot(ho, Wo3_ref[...].astype(bf16), preferred_element_type=f32)
    aud = jnp.dot(ha, Wa3_ref[...].astype(bf16), preferred_element_type=f32)
    sdf_ref[0] = (sdf + bgs_ref[...]) * w[0:1, 0:1]
    img_ref[0] = jax.nn.sigmoid(img + bo3_ref[...]) * w[0:1, 1:2]
    aud_ref[0] = jnp.tanh(aud + ba3_ref[...]) * w[0:1, 2:3]


@functools.partial(jax.jit, static_argnames=("interpret",))
def kernel(user_intent, coords_3d, coords_2d, coords_1d,
           W1, b1, W2, b2, Wr, br, Wt, bt,
           Wg1, bg1, Wg2, bg2, Wgs, bgs, Wgc, bgc,
           Wo1, bo1, Wo2, bo2, Wo3, bo3,
           Wa1, ba1, Wa2, ba2, Wa3, ba3, interpret=False):
    B, N, _ = coords_3d.shape
    GD = user_intent.shape[1]
    LD = Wt.shape[1]
    H = Wg2.shape[0]
    TN = 2048
    S = N // TN
    G = B * S
    f32 = jnp.float32
    bf16 = jnp.bfloat16

    def const1(shape):
        return pl.BlockSpec(shape, lambda: tuple(0 for _ in shape))

    w, bias768, wfirst, wg2b, wo2b, wa2b = pl.pallas_call(
        functools.partial(_router_body, H),
        grid=(),
        in_specs=[
            const1((B, GD)),
            const1((GD, GD)), const1((1, GD)),
            const1((GD, GD)), const1((1, GD)),
            const1((GD, 3)), const1((1, 3)),
            const1((GD, LD)), const1((1, LD)),
            const1((3 + LD, H)), const1((1, H)),
            const1((2 + LD, H)), const1((1, H)),
            const1((1 + LD, H)), const1((1, H)),
            const1((H, H)), const1((H, H)), const1((H, H)),
        ],
        out_specs=[
            const1((B, 3)), const1((B, 3 * H)), const1((8, 3 * H)),
            const1((H, H)), const1((H, H)), const1((H, H)),
        ],
        out_shape=(
            jax.ShapeDtypeStruct((B, 3), f32),
            jax.ShapeDtypeStruct((B, 3 * H), f32),
            jax.ShapeDtypeStruct((8, 3 * H), bf16),
            jax.ShapeDtypeStruct((H, H), bf16),
            jax.ShapeDtypeStruct((H, H), bf16),
            jax.ShapeDtypeStruct((H, H), bf16),
        ),
        interpret=interpret,
    )(user_intent, W1, b1.reshape(1, GD), W2, b2.reshape(1, GD),
      Wr, br.reshape(1, 3), Wt, bt.reshape(1, LD),
      Wg1, bg1.reshape(1, H), Wo1, bo1.reshape(1, H), Wa1, ba1.reshape(1, H),
      Wg2, Wo2, Wa2)

    def const(shape):
        return pl.BlockSpec(shape, lambda b: tuple(0 for _ in shape))

    sdf, img, aud = pl.pallas_call(
        functools.partial(_expert_body, TN, H, S),
        grid=(G,),
        in_specs=[
            const((B, 3)),
            const((B, 3 * H)),
            const((8, 3 * H)),
            const((H, H)), const((H, H)), const((H, H)),
            const((1, H)), const((1, H)), const((1, H)),
            const((H, 1)), const((1, 1)),
            const((H, 3)), const((1, 3)),
            const((H, 1)), const((1, 1)),
            pl.BlockSpec((1, TN, 3), lambda b: (b, 0, 0)),
            pl.BlockSpec((1, TN, 2), lambda b: (b, 0, 0)),
            pl.BlockSpec((1, TN, 1), lambda b: (b, 0, 0)),
        ],
        out_specs=[
            pl.BlockSpec((1, TN, 1), lambda b: (b, 0, 0)),
            pl.BlockSpec((1, TN, 3), lambda b: (b, 0, 0)),
            pl.BlockSpec((1, TN, 1), lambda b: (b, 0, 0)),
        ],
        out_shape=(
            jax.ShapeDtypeStruct((G, TN, 1), f32),
            jax.ShapeDtypeStruct((G, TN, 3), f32),
            jax.ShapeDtypeStruct((G, TN, 1), f32),
        ),
        compiler_params=pltpu.CompilerParams(
            dimension_semantics=("parallel",)),
        interpret=interpret,
    )(w, bias768, wfirst, wg2b, wo2b, wa2b,
      bg2.reshape(1, H), bo2.reshape(1, H), ba2.reshape(1, H),
      Wgs, bgs.reshape(1, 1), Wo3, bo3.reshape(1, 3), Wa3, ba3.reshape(1, 1),
      coords_3d.reshape(G, TN, 3), coords_2d.reshape(G, TN, 2),
      coords_1d.reshape(G, TN, 1))

    return (w, sdf.reshape(B, N, 1), img.reshape(B, N, 3),
            aud.reshape(B, N, 1))
